# baseline (speedup 1.0000x reference)
import functools

import jax
import jax.numpy as jnp
from jax import lax
from jax.experimental import pallas as pl
from jax.experimental.pallas import tpu as pltpu
from jax.experimental.pallas import tpu_sc as plsc

N = 50000
NPAD = 51200
JB = 2048
NJB = NPAD // JB
NI = 20
NBINS = 512
DUMP = 120 * NBINS
HWORDS = 121 * NBINS
HROW = 128 * NBINS
BIGENC = 1 << 30

_NT = 16
_W = NI * NPAD // 2
_CHUNK = 4000
_NCHUNK = _W // _NT // _CHUNK


def _stats_body(pts_ref, emb_ref, t_ref, sem_ref, sums_ref, x_ref, enc_ref):
    jstep = pl.program_id(1)
    emb = pts_ref[...] + emb_ref[...]
    t = t_ref[...]
    sem = sem_ref[...]

    jglob = jstep * JB + lax.broadcasted_iota(jnp.int32, (1, JB), 1)
    inb = jglob < N
    jcol = jstep * JB + lax.broadcasted_iota(jnp.int32, (JB, 3), 0)
    emb = jnp.where(jcol < N, emb, 0.0)
    irow = lax.broadcasted_iota(jnp.int32, (32, JB), 0)
    ohb = (t == irow) & inb
    oh = ohb.astype(jnp.float32)
    ohsem = ((sem == lax.broadcasted_iota(jnp.int32, (8, JB), 0)) & inb
             ).astype(jnp.float32)

    sums_p = lax.dot_general(oh, emb, (((1,), (0,)), ((), ())),
                             preferred_element_type=jnp.float32)
    x_p = lax.dot_general(oh, ohsem, (((1,), (1,)), ((), ())),
                          preferred_element_type=jnp.float32)

    encv = jglob * 8 + sem
    enc = jnp.where(ohb, jnp.broadcast_to(encv, (32, JB)), BIGENC)
    enc_p = jnp.broadcast_to(jnp.min(enc, axis=1, keepdims=True), (32, 8))

    @pl.when(jstep == 0)
    def _():
        sums_ref[...] = jnp.zeros_like(sums_ref)
        x_ref[...] = jnp.zeros_like(x_ref)
        enc_ref[...] = jnp.full_like(enc_ref, BIGENC)

    sums_ref[...] += sums_p
    x_ref[...] += x_p
    enc_ref[...] = jnp.minimum(enc_ref[...], enc_p)


def _idx_body(pts_ref, emb_ref, t_ref, sem_ref, c_ref, out_ref):
    jstep = pl.program_id(1)
    emb = pts_ref[...] + emb_ref[...]
    t = t_ref[...]
    sem = sem_ref[...]
    cmat = c_ref[:, 0:3]

    irow = lax.broadcasted_iota(jnp.int32, (32, JB), 0)
    jlane = lax.broadcasted_iota(jnp.int32, (32, JB), 1)
    dlane = (jlane & 15) + (((jlane >> 10) & 1) << 4)

    cross = lax.dot_general(cmat, emb, (((1,), (1,)), ((), ())),
                            preferred_element_type=jnp.float32)
    nj = lax.dot_general(jnp.ones((32, 3), jnp.float32), emb * emb,
                         (((1,), (1,)), ((), ())),
                         preferred_element_type=jnp.float32)
    ni = jnp.sum(cmat * cmat, axis=1, keepdims=True)
    d2 = ni - 2.0 * cross + nj
    m = jnp.exp(-d2)
    yb = t == irow
    e = jnp.where(yb, 1.0 - m, m)
    k = jnp.clip(jnp.floor(e * NBINS).astype(jnp.int32), 0, NBINS - 1)
    row = (jnp.where(yb, 3, 0) + (sem - 1)) * NI + irow
    idx = row * NBINS + k
    bad = ((sem < 1) | (sem > 3) | (k == 0) | (irow >= NI)
           | (jstep * JB + jlane >= N))
    idx = jnp.where(bad, DUMP + dlane, idx)
    lo = idx[:, 0:JB // 2]
    hi = idx[:, JB // 2:JB]
    packed = lo | (hi << 16)
    out_ref[...] = packed[0:NI, :]


def _hist_tile(idx_ref, out_ref, hist_ref, buf_ref, sem0, sem1):
    c = lax.axis_index("c")
    s = lax.axis_index("s")
    tile = c * _NT + s
    base = c * _W + s * (_NCHUNK * _CHUNK)
    zeros = jnp.zeros((16,), jnp.int32)
    ones = jnp.ones((16,), jnp.int32)
    sems = (sem0, sem1)

    def _start(ch, u):
        pltpu.async_copy(
            idx_ref.at[pl.ds(base + ch * _CHUNK, _CHUNK)],
            buf_ref.at[pl.ds(u * _CHUNK, _CHUNK)], sems[u])

    def _wait(ch, u):
        pltpu.make_async_copy(
            idx_ref.at[pl.ds(base + ch * _CHUNK, _CHUNK)],
            buf_ref.at[pl.ds(u * _CHUNK, _CHUNK)], sems[u]).wait()

    def _scatter(u):
        def _vec(v, _):
            w = buf_ref[pl.ds(u * _CHUNK + v * 16, 16)]
            plsc.addupdate_scatter(hist_ref, [w & 0xFFFF], ones)
            plsc.addupdate_scatter(
                hist_ref, [lax.shift_right_logical(w, 16)], ones)
            return 0

        lax.fori_loop(0, _CHUNK // 16, _vec, 0)

    _start(0, 0)
    _start(1, 1)

    def _zero(i, _):
        for u in range(4):
            hist_ref[pl.ds(i * 64 + u * 16, 16)] = zeros
        return 0

    lax.fori_loop(0, HWORDS // 64, _zero, 0)

    def _outer(g, _):
        for u in range(2):
            ch = g * 2 + u
            _wait(ch, u)
            _scatter(u)
            nxt = jnp.minimum(ch + 2, _NCHUNK - 1)
            _start(nxt, u)
        return 0

    lax.fori_loop(0, _NCHUNK // 2, _outer, 0)
    _wait(_NCHUNK - 1, 0)
    _wait(_NCHUNK - 1, 1)
    pltpu.sync_copy(hist_ref, out_ref.at[pl.ds(tile * HROW, HWORDS)])


def _sc_histogram(idx2):
    mesh = plsc.VectorSubcoreMesh(core_axis_name="c", subcore_axis_name="s")
    return pl.kernel(
        _hist_tile,
        out_type=jax.ShapeDtypeStruct((2 * _NT * HROW,), jnp.int32),
        mesh=mesh,
        compiler_params=pltpu.CompilerParams(needs_layout_passes=False),
        scratch_types=[
            pltpu.VMEM((HWORDS,), jnp.int32),
            pltpu.VMEM((2 * _CHUNK,), jnp.int32),
            pltpu.SemaphoreType.DMA,
            pltpu.SemaphoreType.DMA,
        ],
    )(idx2)


def _loss_body(h_ref, c_ref, out_ref, stat_ref):
    b = pl.program_id(0)
    acc = h_ref[0, 0].astype(jnp.float32)
    for kq in range(1, _NT):
        acc += h_ref[0, kq].astype(jnp.float32)

    r1 = lax.broadcasted_iota(jnp.int32, (NBINS, NBINS), 0)
    r2 = lax.broadcasted_iota(jnp.int32, (NBINS, NBINS), 1)
    tri = (r1 <= r2).astype(jnp.float32)
    suf = lax.dot_general(acc, tri, (((1,), (1,)), ((), ())),
                          preferred_element_type=jnp.float32)
    p1 = lax.broadcasted_iota(jnp.int32, (128, 128), 0)
    p2 = lax.broadcasted_iota(jnp.int32, (128, 128), 1)
    sel = (p2 == p1 + 60).astype(jnp.float32)
    sufp = lax.dot_general(sel, suf, (((1,), (0,)), ((), ())),
                           preferred_element_type=jnp.float32)

    pcnt = c_ref[:, 0:1]
    outpos = c_ref[:, 1:2]
    validc = c_ref[:, 2:3]
    j0 = c_ref[:, 3:4]
    psuf = sufp + outpos
    den = pcnt + suf
    jac = jnp.where(den > 0.0,
                    1.0 - (pcnt - psuf) / jnp.maximum(den, 1.0), 0.0)
    kmask = (lax.broadcasted_iota(jnp.int32, (128, NBINS), 1) >= 1)
    jsum = jnp.sum(jnp.where(kmask, jac, 0.0), axis=1, keepdims=True)
    loss_row = (j0 + jsum) * (1.0 / NBINS)
    tot = jnp.sum(loss_row * validc)
    cntv = jnp.sum(validc)

    @pl.when(b == 0)
    def _():
        stat_ref[0] = tot
        stat_ref[1] = cntv

    @pl.when(b == 1)
    def _():
        tt = stat_ref[0] + tot
        cc = stat_ref[1] + cntv
        val = jnp.where(cc > 0.0, tt / jnp.maximum(cc, 1.0), 0.0)
        out_ref[...] = jnp.full((8, 128), val, jnp.float32)


def kernel(points, target, sem_target, embeddings):
    f32 = jnp.float32
    t3 = target.astype(jnp.int32).reshape(2, 1, N)
    s3 = sem_target.astype(jnp.int32).reshape(2, 1, N)

    pblock = pl.BlockSpec((None, JB, 3), lambda b, j: (b, j, 0))
    iblock = pl.BlockSpec((None, 1, JB), lambda b, j: (b, 0, j))
    tiny = lambda w: pl.BlockSpec((None, 32, w), lambda b, j: (b, 0, 0))
    sums, xtab, enc = pl.pallas_call(
        _stats_body,
        grid=(2, NJB),
        in_specs=[pblock, pblock, iblock, iblock],
        out_specs=[tiny(3), tiny(8), tiny(8)],
        out_shape=[
            jax.ShapeDtypeStruct((2, 32, 3), f32),
            jax.ShapeDtypeStruct((2, 32, 8), f32),
            jax.ShapeDtypeStruct((2, 32, 8), jnp.int32),
        ],
    )(points, embeddings, t3, s3)

    cnt = xtab.sum(axis=2)[:, :NI]
    xci = xtab[:, :NI, 1:4]
    mc = xtab[:, :NI, 1:4].sum(axis=1)
    semfirst = enc[:, :NI, 0] & 7
    centers = sums / jnp.maximum(xtab.sum(axis=2, keepdims=True), 1.0)
    centers = jnp.pad(centers, ((0, 0), (0, 0), (0, 5)))

    prow = jnp.tile(cnt, (1, 3))
    outpos = prow - xci.transpose(0, 2, 1).reshape(2, 60)
    cls = jnp.repeat(jnp.arange(1, 4), NI)[None, :]
    validr = ((jnp.tile(cnt, (1, 3)) > 0)
              & (jnp.tile(semfirst, (1, 3)) == cls)).astype(f32)
    inclass = jnp.repeat(mc, NI, axis=1) - xci.transpose(0, 2, 1).reshape(2, 60)
    j0 = ((prow + inclass) > 0).astype(f32)
    const = jnp.stack([prow, outpos, validr, j0], axis=2)
    const = jnp.pad(const, ((0, 0), (0, 68), (0, 124)))

    idx = pl.pallas_call(
        _idx_body,
        grid=(2, NJB),
        in_specs=[pblock, pblock, iblock, iblock,
                  pl.BlockSpec((None, 32, 8), lambda b, j: (b, 0, 0))],
        out_specs=pl.BlockSpec((None, NI, JB // 2), lambda b, j: (b, 0, j)),
        out_shape=jax.ShapeDtypeStruct((2, NI, NPAD // 2), jnp.int32),
    )(points, embeddings, t3, s3, centers)

    hists = _sc_histogram(idx.reshape(2 * NI * NPAD // 2))
    h4 = hists.reshape(2, _NT, 128, NBINS)

    out = pl.pallas_call(
        _loss_body,
        grid=(2,),
        in_specs=[
            pl.BlockSpec((1, _NT, 128, NBINS), lambda b: (b, 0, 0, 0)),
            pl.BlockSpec((None, 128, 128), lambda b: (b, 0, 0)),
        ],
        out_specs=pl.BlockSpec((8, 128), lambda b: (0, 0)),
        out_shape=jax.ShapeDtypeStruct((8, 128), f32),
        scratch_shapes=[
            pltpu.SMEM((2,), f32),
        ],
    )(h4, const)
    return out[0, 0].reshape(())


# speedup vs baseline: 1.6340x; 1.6340x over previous
import functools

import jax
import jax.numpy as jnp
from jax import lax
from jax.experimental import pallas as pl
from jax.experimental.pallas import tpu as pltpu
from jax.experimental.pallas import tpu_sc as plsc

N = 50000
NPAD = 51200
JB = 2048
NJB = NPAD // JB
NI = 20
NBINS = 512
DUMP = 120 * NBINS
HWORDS = 121 * NBINS
HROW = 128 * NBINS
BIGENC = 1 << 30

_NT = 16
_W = NI * NPAD // 2
_CHUNK = 4000
_NCHUNK = _W // _NT // _CHUNK


def _stats_body(pts_ref, emb_ref, t_ref, sem_ref, sums_ref, x_ref, enc_ref):
    jstep = pl.program_id(1)
    emb = pts_ref[...] + emb_ref[...]
    t = t_ref[...]
    sem = sem_ref[...]

    irow = lax.broadcasted_iota(jnp.int32, (32, JB), 0)
    ohb = t == irow
    oh = ohb.astype(jnp.float32)
    ohsem = (sem == lax.broadcasted_iota(jnp.int32, (8, JB), 0)
             ).astype(jnp.float32)

    sums_p = lax.dot_general(oh, emb, (((1,), (1,)), ((), ())),
                             preferred_element_type=jnp.float32)
    x_p = lax.dot_general(oh, ohsem, (((1,), (1,)), ((), ())),
                          preferred_element_type=jnp.float32)

    jglob = jstep * JB + lax.broadcasted_iota(jnp.int32, (1, JB), 1)
    encv = jglob * 8 + sem
    enc = jnp.where(ohb, jnp.broadcast_to(encv, (32, JB)), BIGENC)
    enc_p = jnp.broadcast_to(jnp.min(enc, axis=1, keepdims=True), (32, 8))

    @pl.when(jstep == 0)
    def _():
        sums_ref[...] = jnp.zeros_like(sums_ref)
        x_ref[...] = jnp.zeros_like(x_ref)
        enc_ref[...] = jnp.full_like(enc_ref, BIGENC)

    sums_ref[...] += sums_p
    x_ref[...] += x_p
    enc_ref[...] = jnp.minimum(enc_ref[...], enc_p)


def _idx_body(pts_ref, emb_ref, t_ref, sem_ref, c_ref, out_ref):
    emb = pts_ref[...] + emb_ref[...]
    t = t_ref[...]
    sem = sem_ref[...]
    x = emb[0:1, :]
    y = emb[1:2, :]
    z = emb[2:3, :]
    cx = c_ref[:, 0:1]
    cy = c_ref[:, 1:2]
    cz = c_ref[:, 2:3]

    irow = lax.broadcasted_iota(jnp.int32, (32, JB), 0)
    jlane = lax.broadcasted_iota(jnp.int32, (32, JB), 1)
    dlane = (jlane & 15) + (((jlane >> 10) & 1) << 4)

    d2 = (x - cx) ** 2 + (y - cy) ** 2 + (z - cz) ** 2
    m = jnp.exp(-d2)
    yb = t == irow
    e = jnp.where(yb, 1.0 - m, m)
    k = jnp.clip(jnp.floor(e * NBINS).astype(jnp.int32), 0, NBINS - 1)
    row = (jnp.where(yb, 3, 0) + (sem - 1)) * NI + irow
    idx = row * NBINS + k
    bad = (sem < 1) | (sem > 3) | (k == 0) | (t < 0) | (irow >= NI)
    idx = jnp.where(bad, DUMP + dlane, idx)
    lo = idx[:, 0:JB // 2]
    hi = idx[:, JB // 2:JB]
    packed = lo | (hi << 16)
    out_ref[...] = packed[0:NI, :]


def _hist_tile(idx_ref, out_ref, hist_ref, buf_ref, sem0, sem1):
    c = lax.axis_index("c")
    s = lax.axis_index("s")
    tile = c * _NT + s
    base = c * _W + s * (_NCHUNK * _CHUNK)
    zeros = jnp.zeros((16,), jnp.int32)
    ones = jnp.ones((16,), jnp.int32)
    sems = (sem0, sem1)

    def _start(ch, u):
        pltpu.async_copy(
            idx_ref.at[pl.ds(base + ch * _CHUNK, _CHUNK)],
            buf_ref.at[pl.ds(u * _CHUNK, _CHUNK)], sems[u])

    def _wait(ch, u):
        pltpu.make_async_copy(
            idx_ref.at[pl.ds(base + ch * _CHUNK, _CHUNK)],
            buf_ref.at[pl.ds(u * _CHUNK, _CHUNK)], sems[u]).wait()

    def _scatter(u):
        def _vec(v, _):
            w = buf_ref[pl.ds(u * _CHUNK + v * 16, 16)]
            plsc.addupdate_scatter(hist_ref, [w & 0xFFFF], ones)
            plsc.addupdate_scatter(
                hist_ref, [lax.shift_right_logical(w, 16)], ones)
            return 0

        lax.fori_loop(0, _CHUNK // 16, _vec, 0)

    _start(0, 0)
    _start(1, 1)

    def _zero(i, _):
        for u in range(4):
            hist_ref[pl.ds(i * 64 + u * 16, 16)] = zeros
        return 0

    lax.fori_loop(0, HWORDS // 64, _zero, 0)

    def _outer(g, _):
        for u in range(2):
            ch = g * 2 + u
            _wait(ch, u)
            _scatter(u)
            nxt = jnp.minimum(ch + 2, _NCHUNK - 1)
            _start(nxt, u)
        return 0

    lax.fori_loop(0, _NCHUNK // 2, _outer, 0)
    _wait(_NCHUNK - 1, 0)
    _wait(_NCHUNK - 1, 1)
    pltpu.sync_copy(hist_ref, out_ref.at[pl.ds(tile * HROW, HWORDS)])


def _sc_histogram(idx2):
    mesh = plsc.VectorSubcoreMesh(core_axis_name="c", subcore_axis_name="s")
    return pl.kernel(
        _hist_tile,
        out_type=jax.ShapeDtypeStruct((2 * _NT * HROW,), jnp.int32),
        mesh=mesh,
        compiler_params=pltpu.CompilerParams(needs_layout_passes=False),
        scratch_types=[
            pltpu.VMEM((HWORDS,), jnp.int32),
            pltpu.VMEM((2 * _CHUNK,), jnp.int32),
            pltpu.SemaphoreType.DMA,
            pltpu.SemaphoreType.DMA,
        ],
    )(idx2)


def _loss_body(h_ref, c_ref, out_ref, stat_ref):
    b = pl.program_id(0)
    acc = h_ref[0, 0].astype(jnp.float32)
    for kq in range(1, _NT):
        acc += h_ref[0, kq].astype(jnp.float32)

    r1 = lax.broadcasted_iota(jnp.int32, (NBINS, NBINS), 0)
    r2 = lax.broadcasted_iota(jnp.int32, (NBINS, NBINS), 1)
    tri = (r1 <= r2).astype(jnp.float32)
    suf = lax.dot_general(acc, tri, (((1,), (1,)), ((), ())),
                          preferred_element_type=jnp.float32)
    p1 = lax.broadcasted_iota(jnp.int32, (128, 128), 0)
    p2 = lax.broadcasted_iota(jnp.int32, (128, 128), 1)
    sel = (p2 == p1 + 60).astype(jnp.float32)
    sufp = lax.dot_general(sel, suf, (((1,), (0,)), ((), ())),
                           preferred_element_type=jnp.float32)

    pcnt = c_ref[:, 0:1]
    outpos = c_ref[:, 1:2]
    validc = c_ref[:, 2:3]
    j0 = c_ref[:, 3:4]
    psuf = sufp + outpos
    den = pcnt + suf
    jac = jnp.where(den > 0.0,
                    1.0 - (pcnt - psuf) / jnp.maximum(den, 1.0), 0.0)
    kmask = (lax.broadcasted_iota(jnp.int32, (128, NBINS), 1) >= 1)
    jsum = jnp.sum(jnp.where(kmask, jac, 0.0), axis=1, keepdims=True)
    loss_row = (j0 + jsum) * (1.0 / NBINS)
    tot = jnp.sum(loss_row * validc)
    cntv = jnp.sum(validc)

    @pl.when(b == 0)
    def _():
        stat_ref[0] = tot
        stat_ref[1] = cntv

    @pl.when(b == 1)
    def _():
        tt = stat_ref[0] + tot
        cc = stat_ref[1] + cntv
        val = jnp.where(cc > 0.0, tt / jnp.maximum(cc, 1.0), 0.0)
        out_ref[...] = jnp.full((8, 128), val, jnp.float32)


def kernel(points, target, sem_target, embeddings):
    f32 = jnp.float32
    ptsT = jnp.pad(points.transpose(0, 2, 1), ((0, 0), (0, 5), (0, NPAD - N)))
    embT = jnp.pad(embeddings.transpose(0, 2, 1),
                   ((0, 0), (0, 5), (0, NPAD - N)))
    t3 = jnp.pad(target[..., 0].astype(jnp.int32), ((0, 0), (0, NPAD - N)),
                 constant_values=-1).reshape(2, 1, NPAD)
    s3 = jnp.pad(sem_target[..., 0].astype(jnp.int32),
                 ((0, 0), (0, NPAD - N))).reshape(2, 1, NPAD)

    vblock = lambda r: pl.BlockSpec((None, r, JB), lambda b, j: (b, 0, j))
    tiny = lambda: pl.BlockSpec((None, 32, 8), lambda b, j: (b, 0, 0))
    sums, xtab, enc = pl.pallas_call(
        _stats_body,
        grid=(2, NJB),
        in_specs=[vblock(8), vblock(8), vblock(1), vblock(1)],
        out_specs=[tiny(), tiny(), tiny()],
        out_shape=[
            jax.ShapeDtypeStruct((2, 32, 8), f32),
            jax.ShapeDtypeStruct((2, 32, 8), f32),
            jax.ShapeDtypeStruct((2, 32, 8), jnp.int32),
        ],
    )(ptsT, embT, t3, s3)

    cnt = xtab.sum(axis=2)[:, :NI]
    xci = xtab[:, :NI, 1:4]
    mc = xtab[:, :NI, 1:4].sum(axis=1)
    semfirst = enc[:, :NI, 0] & 7
    centers = sums[:, :, :3] / jnp.maximum(
        xtab.sum(axis=2, keepdims=True), 1.0)
    centers = jnp.pad(centers, ((0, 0), (0, 0), (0, 125)))

    prow = jnp.tile(cnt, (1, 3))
    outpos = prow - xci.transpose(0, 2, 1).reshape(2, 60)
    cls = jnp.repeat(jnp.arange(1, 4), NI)[None, :]
    validr = ((jnp.tile(cnt, (1, 3)) > 0)
              & (jnp.tile(semfirst, (1, 3)) == cls)).astype(f32)
    inclass = jnp.repeat(mc, NI, axis=1) - xci.transpose(0, 2, 1).reshape(2, 60)
    j0 = ((prow + inclass) > 0).astype(f32)
    const = jnp.stack([prow, outpos, validr, j0], axis=2)
    const = jnp.pad(const, ((0, 0), (0, 68), (0, 124)))

    idx = pl.pallas_call(
        _idx_body,
        grid=(2, NJB),
        in_specs=[vblock(8), vblock(8), vblock(1), vblock(1),
                  pl.BlockSpec((None, 32, 128), lambda b, j: (b, 0, 0))],
        out_specs=pl.BlockSpec((None, NI, JB // 2), lambda b, j: (b, 0, j)),
        out_shape=jax.ShapeDtypeStruct((2, NI, NPAD // 2), jnp.int32),
    )(ptsT, embT, t3, s3, centers)

    hists = _sc_histogram(idx.reshape(2 * NI * NPAD // 2))
    h4 = hists.reshape(2, _NT, 128, NBINS)

    out = pl.pallas_call(
        _loss_body,
        grid=(2,),
        in_specs=[
            pl.BlockSpec((1, _NT, 128, NBINS), lambda b: (b, 0, 0, 0)),
            pl.BlockSpec((None, 128, 128), lambda b: (b, 0, 0)),
        ],
        out_specs=pl.BlockSpec((8, 128), lambda b: (0, 0)),
        out_shape=jax.ShapeDtypeStruct((8, 128), f32),
        scratch_shapes=[
            pltpu.SMEM((2,), f32),
        ],
    )(h4, const)
    return out[0, 0].reshape(())


# speedup vs baseline: 1.6369x; 1.0018x over previous
import functools

import jax
import jax.numpy as jnp
from jax import lax
from jax.experimental import pallas as pl
from jax.experimental.pallas import tpu as pltpu
from jax.experimental.pallas import tpu_sc as plsc

N = 50000
NPAD = 51200
JB = 2048
NJB = NPAD // JB
NI = 20
NBINS = 512
DUMP = 120 * NBINS
HWORDS = 121 * NBINS
HROW = 128 * NBINS
BIGENC = 1 << 30

_NT = 16
_W = NI * NPAD // 2
_CHUNK = 4000
_NCHUNK = _W // _NT // _CHUNK


def _ab_body(pts_ref, emb_ref, t_ref, sem_ref,
             sums_ref, x_ref, enc_ref, idx_ref, c_scr):
    phase = pl.program_id(1)
    jstep = pl.program_id(2)
    emb = pts_ref[...] + emb_ref[...]
    t = t_ref[...]
    sem = sem_ref[...]
    irow = lax.broadcasted_iota(jnp.int32, (32, JB), 0)

    @pl.when(phase == 0)
    def _():
        ohb = t == irow
        oh = ohb.astype(jnp.float32)
        ohsem = (sem == lax.broadcasted_iota(jnp.int32, (8, JB), 0)
                 ).astype(jnp.float32)

        sums_p = lax.dot_general(oh, emb, (((1,), (1,)), ((), ())),
                                 preferred_element_type=jnp.float32)
        x_p = lax.dot_general(oh, ohsem, (((1,), (1,)), ((), ())),
                              preferred_element_type=jnp.float32)

        jglob = jstep * JB + lax.broadcasted_iota(jnp.int32, (1, JB), 1)
        encv = jglob * 8 + sem
        enc = jnp.where(ohb, jnp.broadcast_to(encv, (32, JB)), BIGENC)
        enc_p = jnp.broadcast_to(jnp.min(enc, axis=1, keepdims=True), (32, 8))

        @pl.when(jstep == 0)
        def _():
            sums_ref[...] = jnp.zeros_like(sums_ref)
            x_ref[...] = jnp.zeros_like(x_ref)
            enc_ref[...] = jnp.full_like(enc_ref, BIGENC)

        sums_ref[...] += sums_p
        x_ref[...] += x_p
        enc_ref[...] = jnp.minimum(enc_ref[...], enc_p)

    @pl.when(phase == 1)
    def _():
        @pl.when(jstep == 0)
        def _():
            cnt = jnp.sum(x_ref[...], axis=1, keepdims=True)
            c_scr[...] = sums_ref[...] / jnp.maximum(cnt, 1.0)

        xx = emb[0:1, :]
        yy = emb[1:2, :]
        zz = emb[2:3, :]
        cx = c_scr[:, 0:1]
        cy = c_scr[:, 1:2]
        cz = c_scr[:, 2:3]

        jlane = lax.broadcasted_iota(jnp.int32, (32, JB), 1)
        dlane = (jlane & 15) + (((jlane >> 10) & 1) << 4)

        d2 = (xx - cx) ** 2 + (yy - cy) ** 2 + (zz - cz) ** 2
        m = jnp.exp(-d2)
        yb = t == irow
        e = jnp.where(yb, 1.0 - m, m)
        k = jnp.clip(jnp.floor(e * NBINS).astype(jnp.int32), 0, NBINS - 1)
        row = (jnp.where(yb, 3, 0) + (sem - 1)) * NI + irow
        idx = row * NBINS + k
        bad = (sem < 1) | (sem > 3) | (k == 0) | (t < 0) | (irow >= NI)
        idx = jnp.where(bad, DUMP + dlane, idx)
        lo = idx[:, 0:JB // 2]
        hi = idx[:, JB // 2:JB]
        packed = lo | (hi << 16)
        idx_ref[...] = packed[0:NI, :]


def _hist_tile(idx_ref, out_ref, hist_ref, buf_ref, sem0, sem1):
    c = lax.axis_index("c")
    s = lax.axis_index("s")
    tile = c * _NT + s
    base = c * _W + s * (_NCHUNK * _CHUNK)
    zeros = jnp.zeros((16,), jnp.int32)
    ones = jnp.ones((16,), jnp.int32)
    sems = (sem0, sem1)

    def _start(ch, u):
        pltpu.async_copy(
            idx_ref.at[pl.ds(base + ch * _CHUNK, _CHUNK)],
            buf_ref.at[pl.ds(u * _CHUNK, _CHUNK)], sems[u])

    def _wait(ch, u):
        pltpu.make_async_copy(
            idx_ref.at[pl.ds(base + ch * _CHUNK, _CHUNK)],
            buf_ref.at[pl.ds(u * _CHUNK, _CHUNK)], sems[u]).wait()

    def _scatter(u):
        def _vec(v, _):
            w = buf_ref[pl.ds(u * _CHUNK + v * 16, 16)]
            plsc.addupdate_scatter(hist_ref, [w & 0xFFFF], ones)
            plsc.addupdate_scatter(
                hist_ref, [lax.shift_right_logical(w, 16)], ones)
            return 0

        lax.fori_loop(0, _CHUNK // 16, _vec, 0)

    _start(0, 0)
    _start(1, 1)

    def _zero(i, _):
        for u in range(4):
            hist_ref[pl.ds(i * 64 + u * 16, 16)] = zeros
        return 0

    lax.fori_loop(0, HWORDS // 64, _zero, 0)

    def _outer(g, _):
        for u in range(2):
            ch = g * 2 + u
            _wait(ch, u)
            _scatter(u)
            nxt = jnp.minimum(ch + 2, _NCHUNK - 1)
            _start(nxt, u)
        return 0

    lax.fori_loop(0, _NCHUNK // 2, _outer, 0)
    _wait(_NCHUNK - 1, 0)
    _wait(_NCHUNK - 1, 1)
    pltpu.sync_copy(hist_ref, out_ref.at[pl.ds(tile * HROW, HWORDS)])


def _sc_histogram(idx2):
    mesh = plsc.VectorSubcoreMesh(core_axis_name="c", subcore_axis_name="s")
    return pl.kernel(
        _hist_tile,
        out_type=jax.ShapeDtypeStruct((2 * _NT * HROW,), jnp.int32),
        mesh=mesh,
        compiler_params=pltpu.CompilerParams(needs_layout_passes=False),
        scratch_types=[
            pltpu.VMEM((HWORDS,), jnp.int32),
            pltpu.VMEM((2 * _CHUNK,), jnp.int32),
            pltpu.SemaphoreType.DMA,
            pltpu.SemaphoreType.DMA,
        ],
    )(idx2)


def _loss_body(h_ref, c_ref, out_ref, stat_ref):
    b = pl.program_id(0)
    acc = h_ref[0, 0].astype(jnp.float32)
    for kq in range(1, _NT):
        acc += h_ref[0, kq].astype(jnp.float32)

    r1 = lax.broadcasted_iota(jnp.int32, (NBINS, NBINS), 0)
    r2 = lax.broadcasted_iota(jnp.int32, (NBINS, NBINS), 1)
    tri = (r1 <= r2).astype(jnp.float32)
    suf = lax.dot_general(acc, tri, (((1,), (1,)), ((), ())),
                          preferred_element_type=jnp.float32)
    p1 = lax.broadcasted_iota(jnp.int32, (128, 128), 0)
    p2 = lax.broadcasted_iota(jnp.int32, (128, 128), 1)
    sel = (p2 == p1 + 60).astype(jnp.float32)
    sufp = lax.dot_general(sel, suf, (((1,), (0,)), ((), ())),
                           preferred_element_type=jnp.float32)

    pcnt = c_ref[:, 0:1]
    outpos = c_ref[:, 1:2]
    validc = c_ref[:, 2:3]
    j0 = c_ref[:, 3:4]
    psuf = sufp + outpos
    den = pcnt + suf
    jac = jnp.where(den > 0.0,
                    1.0 - (pcnt - psuf) / jnp.maximum(den, 1.0), 0.0)
    kmask = (lax.broadcasted_iota(jnp.int32, (128, NBINS), 1) >= 1)
    jsum = jnp.sum(jnp.where(kmask, jac, 0.0), axis=1, keepdims=True)
    loss_row = (j0 + jsum) * (1.0 / NBINS)
    tot = jnp.sum(loss_row * validc)
    cntv = jnp.sum(validc)

    @pl.when(b == 0)
    def _():
        stat_ref[0] = tot
        stat_ref[1] = cntv

    @pl.when(b == 1)
    def _():
        tt = stat_ref[0] + tot
        cc = stat_ref[1] + cntv
        val = jnp.where(cc > 0.0, tt / jnp.maximum(cc, 1.0), 0.0)
        out_ref[...] = jnp.full((8, 128), val, jnp.float32)


def kernel(points, target, sem_target, embeddings):
    f32 = jnp.float32
    ptsT = jnp.pad(points.transpose(0, 2, 1), ((0, 0), (0, 5), (0, NPAD - N)))
    embT = jnp.pad(embeddings.transpose(0, 2, 1),
                   ((0, 0), (0, 5), (0, NPAD - N)))
    t3 = jnp.pad(target[..., 0].astype(jnp.int32), ((0, 0), (0, NPAD - N)),
                 constant_values=-1).reshape(2, 1, NPAD)
    s3 = jnp.pad(sem_target[..., 0].astype(jnp.int32),
                 ((0, 0), (0, NPAD - N))).reshape(2, 1, NPAD)

    vblock = lambda r: pl.BlockSpec((None, r, JB), lambda b, p, j: (b, 0, j))
    tiny = lambda: pl.BlockSpec((None, 32, 8), lambda b, p, j: (b, 0, 0))
    sums, xtab, enc, idx = pl.pallas_call(
        _ab_body,
        grid=(2, 2, NJB),
        in_specs=[vblock(8), vblock(8), vblock(1), vblock(1)],
        out_specs=[tiny(), tiny(), tiny(),
                   pl.BlockSpec((None, NI, JB // 2), lambda b, p, j: (b, 0, j))],
        out_shape=[
            jax.ShapeDtypeStruct((2, 32, 8), f32),
            jax.ShapeDtypeStruct((2, 32, 8), f32),
            jax.ShapeDtypeStruct((2, 32, 8), jnp.int32),
            jax.ShapeDtypeStruct((2, NI, NPAD // 2), jnp.int32),
        ],
        scratch_shapes=[pltpu.VMEM((32, 8), f32)],
    )(ptsT, embT, t3, s3)

    cnt = xtab.sum(axis=2)[:, :NI]
    xci = xtab[:, :NI, 1:4]
    mc = xtab[:, :NI, 1:4].sum(axis=1)
    semfirst = enc[:, :NI, 0] & 7

    prow = jnp.tile(cnt, (1, 3))
    outpos = prow - xci.transpose(0, 2, 1).reshape(2, 60)
    cls = jnp.repeat(jnp.arange(1, 4), NI)[None, :]
    validr = ((jnp.tile(cnt, (1, 3)) > 0)
              & (jnp.tile(semfirst, (1, 3)) == cls)).astype(f32)
    inclass = jnp.repeat(mc, NI, axis=1) - xci.transpose(0, 2, 1).reshape(2, 60)
    j0 = ((prow + inclass) > 0).astype(f32)
    const = jnp.stack([prow, outpos, validr, j0], axis=2)
    const = jnp.pad(const, ((0, 0), (0, 68), (0, 124)))

    hists = _sc_histogram(idx.reshape(2 * NI * NPAD // 2))
    h4 = hists.reshape(2, _NT, 128, NBINS)

    out = pl.pallas_call(
        _loss_body,
        grid=(2,),
        in_specs=[
            pl.BlockSpec((1, _NT, 128, NBINS), lambda b: (b, 0, 0, 0)),
            pl.BlockSpec((None, 128, 128), lambda b: (b, 0, 0)),
        ],
        out_specs=pl.BlockSpec((8, 128), lambda b: (0, 0)),
        out_shape=jax.ShapeDtypeStruct((8, 128), f32),
        scratch_shapes=[
            pltpu.SMEM((2,), f32),
        ],
    )(h4, const)
    return out[0, 0].reshape(())
